# Initial kernel scaffold; baseline (speedup 1.0000x reference)
import jax
import jax.numpy as jnp
from jax.experimental import pallas as pl


def kernel(x, thresholds, mapping1, luts1, mapping2, luts2, log_alpha, beta):
    raise NotImplementedError("write your pallas kernel here")


# speedup vs baseline: 352.5245x; 352.5245x over previous
import functools

import jax
import jax.numpy as jnp
from jax import lax
from jax.experimental import pallas as pl
from jax.experimental.pallas import tpu as pltpu
from jax.experimental.pallas import tpu_sc as plsc

_OBS = 128
_BITS = 64
_ACT = 8
_N = 6
_SIZE = 1200
_EPS = 1e-6
_BATCH = 4096
_GROUP = _SIZE // _ACT
_GPAD = 160
_NC = 2
_NS = 16
_NW = _NC * _NS
_BPW = _BATCH // _NW
_CHUNKS = _SIZE // 16


def _wnn_body(x_hbm, d1_hbm, v1_hbm, m2_hbm, lo1_hbm, hi1_hbm, lo2_hbm,
              hi2_hbm, sidx_hbm, la_hbm, be_hbm, out_hbm,
              x_v, d1_v, v1_v, m2_v, lo1_v, hi1_v, lo2_v, hi2_v, sidx_v,
              la_v, be_v, b1_v, b2_v, y_v):
    wid = lax.axis_index("s") * _NC + lax.axis_index("c")
    base = wid * _BPW

    pltpu.sync_copy(x_hbm.at[pl.ds(base * _OBS, _BPW * _OBS)], x_v)
    pltpu.sync_copy(d1_hbm, d1_v)
    pltpu.sync_copy(v1_hbm, v1_v)
    pltpu.sync_copy(m2_hbm, m2_v)
    pltpu.sync_copy(lo1_hbm, lo1_v)
    pltpu.sync_copy(hi1_hbm, hi1_v)
    pltpu.sync_copy(lo2_hbm, lo2_v)
    pltpu.sync_copy(hi2_hbm, hi2_v)
    pltpu.sync_copy(sidx_hbm, sidx_v)
    pltpu.sync_copy(la_hbm, la_v)
    pltpu.sync_copy(be_hbm, be_v)

    iota = lax.iota(jnp.int32, 16)
    zero16 = jnp.zeros((16,), jnp.int32)

    def zero_body(i, carry):
        b2_v[pl.ds(i * 16, 16)] = zero16
        return carry

    lax.fori_loop(0, (_ACT * _GPAD) // 16, zero_body, 0)

    ea16 = jnp.exp(la_v[...])
    be16 = be_v[...]

    def row_body(b, carry):
        bs = jnp.full((16,), b, dtype=jnp.int32)
        xoff = b * _OBS

        def l1_chunk(c, carry1):
            off = c * 16
            addr = zero16
            for j in range(_N):
                idx = d1_v[pl.ds(j * _SIZE + off, 16)] + xoff
                xv = plsc.load_gather(x_v, [idx])
                vv = v1_v[pl.ds(j * _SIZE + off, 16)]
                addr = addr | ((xv > vv).astype(jnp.int32) << j)
            word = jnp.where(addr >= 32, hi1_v[pl.ds(off, 16)],
                             lo1_v[pl.ds(off, 16)])
            b1_v[pl.ds(off, 16)] = lax.shift_right_logical(word, addr & 31) & 1
            return carry1

        lax.fori_loop(0, _CHUNKS, l1_chunk, 0)

        def l2_chunk(c, carry2):
            off = c * 16
            addr = zero16
            for j in range(_N):
                idx = m2_v[pl.ds(j * _SIZE + off, 16)]
                bv = plsc.load_gather(b1_v, [idx])
                addr = addr | (bv << j)
            word = jnp.where(addr >= 32, hi2_v[pl.ds(off, 16)],
                             lo2_v[pl.ds(off, 16)])
            bit2 = lax.shift_right_logical(word, addr & 31) & 1
            plsc.store_scatter(b2_v, [sidx_v[pl.ds(off, 16)]], bit2)
            return carry2

        lax.fori_loop(0, _CHUNKS, l2_chunk, 0)

        gv = zero16
        for k in range(_ACT):
            acc = zero16
            for cc in range(_GPAD // 16):
                acc = acc + b2_v[pl.ds(k * _GPAD + cc * 16, 16)]
            gv = jnp.where(iota == k, jnp.sum(acc), gv)

        xn = jnp.clip(gv.astype(jnp.float32) / float(_GROUP), _EPS, 1.0 - _EPS)
        y16 = ea16 * (xn - 0.5) + be16
        plsc.store_scatter(y_v, [b * _ACT + iota], y16, mask=iota < _ACT)
        return carry

    lax.fori_loop(0, _BPW, row_body, 0)
    pltpu.sync_copy(y_v, out_hbm.at[pl.ds(base * _ACT, _BPW * _ACT)])


def _pack_sign_masks(luts):
    s = (luts >= 0).astype(jnp.uint32)
    sh = jnp.arange(32, dtype=jnp.uint32)
    lo = jnp.sum(s[:, :32] << sh, axis=1, dtype=jnp.uint32)
    hi = jnp.sum(s[:, 32:] << sh, axis=1, dtype=jnp.uint32)
    return (lax.bitcast_convert_type(lo, jnp.int32),
            lax.bitcast_convert_type(hi, jnp.int32))


@jax.jit
def kernel(x, thresholds, mapping1, luts1, mapping2, luts2, log_alpha, beta):
    thr_flat = thresholds.reshape(-1)
    v1 = thr_flat[mapping1].T.reshape(-1)
    d1 = (mapping1 // _BITS).T.reshape(-1).astype(jnp.int32)
    m2 = mapping2.T.reshape(-1).astype(jnp.int32)
    lo1, hi1 = _pack_sign_masks(luts1)
    lo2, hi2 = _pack_sign_masks(luts2)
    o = jnp.arange(_SIZE, dtype=jnp.int32)
    sidx = o + (_GPAD - _GROUP) * (o // _GROUP)
    la16 = jnp.tile(log_alpha, 2)
    be16 = jnp.tile(beta, 2)

    mesh = plsc.VectorSubcoreMesh(core_axis_name="c", subcore_axis_name="s")
    run = pl.kernel(
        _wnn_body,
        out_type=jax.ShapeDtypeStruct((_BATCH * _ACT,), jnp.float32),
        mesh=mesh,
        compiler_params=pltpu.CompilerParams(needs_layout_passes=False),
        scratch_types=[
            pltpu.VMEM((_BPW * _OBS,), jnp.float32),
            pltpu.VMEM((_N * _SIZE,), jnp.int32),
            pltpu.VMEM((_N * _SIZE,), jnp.float32),
            pltpu.VMEM((_N * _SIZE,), jnp.int32),
            pltpu.VMEM((_SIZE,), jnp.int32),
            pltpu.VMEM((_SIZE,), jnp.int32),
            pltpu.VMEM((_SIZE,), jnp.int32),
            pltpu.VMEM((_SIZE,), jnp.int32),
            pltpu.VMEM((_SIZE,), jnp.int32),
            pltpu.VMEM((16,), jnp.float32),
            pltpu.VMEM((16,), jnp.float32),
            pltpu.VMEM((_SIZE,), jnp.int32),
            pltpu.VMEM((_ACT * _GPAD,), jnp.int32),
            pltpu.VMEM((_BPW * _ACT,), jnp.float32),
        ],
    )
    out = run(x.reshape(-1), d1, v1, m2, lo1, hi1, lo2, hi2, sidx, la16, be16)
    return out.reshape(_BATCH, _ACT)


# speedup vs baseline: 430.4392x; 1.2210x over previous
import functools

import jax
import jax.numpy as jnp
from jax import lax
from jax.experimental import pallas as pl
from jax.experimental.pallas import tpu as pltpu
from jax.experimental.pallas import tpu_sc as plsc

_OBS = 128
_BITS = 64
_ACT = 8
_N = 6
_SIZE = 1200
_EPS = 1e-6
_BATCH = 4096
_GROUP = _SIZE // _ACT
_NC = 2
_NS = 16
_NW = _NC * _NS
_BPW = _BATCH // _NW
_CHUNKS = _SIZE // 16
_RB = 16
_NBLK = _BPW // _RB
_GACC = _ACT * 16


def _wnn_body(x_hbm, d1_hbm, v1_hbm, m2_hbm, lo1_hbm, hi1_hbm, lo2_hbm,
              hi2_hbm, sidx_hbm, la_hbm, be_hbm, out_hbm,
              x_v, d1_v, v1_v, m2_v, lo1_v, hi1_v, lo2_v, hi2_v, sidx_v,
              la_v, be_v, b1_v, gacc_v, y_v):
    wid = lax.axis_index("s") * _NC + lax.axis_index("c")
    base = wid * _BPW

    pltpu.sync_copy(x_hbm.at[pl.ds(base * _OBS, _BPW * _OBS)], x_v)
    pltpu.sync_copy(d1_hbm, d1_v)
    pltpu.sync_copy(v1_hbm, v1_v)
    pltpu.sync_copy(m2_hbm, m2_v)
    pltpu.sync_copy(lo1_hbm, lo1_v)
    pltpu.sync_copy(hi1_hbm, hi1_v)
    pltpu.sync_copy(lo2_hbm, lo2_v)
    pltpu.sync_copy(hi2_hbm, hi2_v)
    pltpu.sync_copy(sidx_hbm, sidx_v)
    pltpu.sync_copy(la_hbm, la_v)
    pltpu.sync_copy(be_hbm, be_v)

    iota = lax.iota(jnp.int32, 16)
    zero16 = jnp.zeros((16,), jnp.int32)

    ea16 = jnp.exp(la_v[...])
    be16 = be_v[...]

    def block_body(blk, carry):
        row0 = blk * _RB

        def zero_body(i, c):
            gacc_v[pl.ds(i * 16, 16)] = zero16
            return c

        lax.fori_loop(0, (_RB * _GACC) // 16, zero_body, 0)

        def l1_chunk(c, carry1):
            off = c * 16
            idxs = [d1_v[pl.ds(j * _SIZE + off, 16)] for j in range(_N)]
            vals = [v1_v[pl.ds(j * _SIZE + off, 16)] for j in range(_N)]
            wlo = lo1_v[pl.ds(off, 16)]
            whi = hi1_v[pl.ds(off, 16)]
            for r in range(_RB):
                xrow = x_v.at[pl.ds((row0 + r) * _OBS, _OBS)]
                addr = zero16
                for j in range(_N):
                    xv = plsc.load_gather(xrow, [idxs[j]])
                    addr = addr | jnp.where(xv > vals[j], 1 << j, 0)
                word = jnp.where(addr >= 32, whi, wlo)
                bit = lax.shift_right_logical(word, addr & 31) & 1
                b1_v[pl.ds(r * _SIZE + off, 16)] = bit
            return carry1

        lax.fori_loop(0, _CHUNKS, l1_chunk, 0)

        def l2_chunk(c, carry2):
            off = c * 16
            idxs = [m2_v[pl.ds(j * _SIZE + off, 16)] for j in range(_N)]
            wlo = lo2_v[pl.ds(off, 16)]
            whi = hi2_v[pl.ds(off, 16)]
            si = sidx_v[pl.ds(off, 16)]
            for r in range(_RB):
                brow = b1_v.at[pl.ds(r * _SIZE, _SIZE)]
                addr = zero16
                for j in range(_N):
                    addr = addr | (plsc.load_gather(brow, [idxs[j]]) << j)
                word = jnp.where(addr >= 32, whi, wlo)
                bit = lax.shift_right_logical(word, addr & 31) & 1
                plsc.addupdate_scatter(gacc_v.at[pl.ds(r * _GACC, _GACC)],
                                       [si], bit)
            return carry2

        lax.fori_loop(0, _CHUNKS, l2_chunk, 0)

        for r in range(_RB):
            gv = zero16
            for k in range(_ACT):
                acc = gacc_v[pl.ds(r * _GACC + k * 16, 16)]
                gv = jnp.where(iota == k, jnp.sum(acc), gv)
            xn = jnp.clip(gv.astype(jnp.float32) / float(_GROUP),
                          _EPS, 1.0 - _EPS)
            y16 = ea16 * (xn - 0.5) + be16
            plsc.store_scatter(y_v, [(row0 + r) * _ACT + iota], y16,
                               mask=iota < _ACT)
        return carry

    lax.fori_loop(0, _NBLK, block_body, 0)
    pltpu.sync_copy(y_v, out_hbm.at[pl.ds(base * _ACT, _BPW * _ACT)])


def _pack_sign_masks(luts):
    s = (luts >= 0).astype(jnp.uint32)
    sh = jnp.arange(32, dtype=jnp.uint32)
    lo = jnp.sum(s[:, :32] << sh, axis=1, dtype=jnp.uint32)
    hi = jnp.sum(s[:, 32:] << sh, axis=1, dtype=jnp.uint32)
    return (lax.bitcast_convert_type(lo, jnp.int32),
            lax.bitcast_convert_type(hi, jnp.int32))


@jax.jit
def kernel(x, thresholds, mapping1, luts1, mapping2, luts2, log_alpha, beta):
    thr_flat = thresholds.reshape(-1)
    v1 = thr_flat[mapping1].T.reshape(-1)
    d1 = (mapping1 // _BITS).T.reshape(-1).astype(jnp.int32)
    m2 = mapping2.T.reshape(-1).astype(jnp.int32)
    lo1, hi1 = _pack_sign_masks(luts1)
    lo2, hi2 = _pack_sign_masks(luts2)
    o = jnp.arange(_SIZE, dtype=jnp.int32)
    sidx = (o // _GROUP) * 16 + (o % 16)
    la16 = jnp.tile(log_alpha, 2)
    be16 = jnp.tile(beta, 2)

    mesh = plsc.VectorSubcoreMesh(core_axis_name="c", subcore_axis_name="s")
    run = pl.kernel(
        _wnn_body,
        out_type=jax.ShapeDtypeStruct((_BATCH * _ACT,), jnp.float32),
        mesh=mesh,
        compiler_params=pltpu.CompilerParams(needs_layout_passes=False),
        scratch_types=[
            pltpu.VMEM((_BPW * _OBS,), jnp.float32),
            pltpu.VMEM((_N * _SIZE,), jnp.int32),
            pltpu.VMEM((_N * _SIZE,), jnp.float32),
            pltpu.VMEM((_N * _SIZE,), jnp.int32),
            pltpu.VMEM((_SIZE,), jnp.int32),
            pltpu.VMEM((_SIZE,), jnp.int32),
            pltpu.VMEM((_SIZE,), jnp.int32),
            pltpu.VMEM((_SIZE,), jnp.int32),
            pltpu.VMEM((_SIZE,), jnp.int32),
            pltpu.VMEM((16,), jnp.float32),
            pltpu.VMEM((16,), jnp.float32),
            pltpu.VMEM((_RB * _SIZE,), jnp.int32),
            pltpu.VMEM((_RB * _GACC,), jnp.int32),
            pltpu.VMEM((_BPW * _ACT,), jnp.float32),
        ],
    )
    out = run(x.reshape(-1), d1, v1, m2, lo1, hi1, lo2, hi2, sidx, la16, be16)
    return out.reshape(_BATCH, _ACT)
